# baseline (speedup 1.0000x reference)
import functools

import jax
import jax.numpy as jnp
from jax import lax
from jax.experimental import pallas as pl
from jax.experimental.pallas import tpu as pltpu
from jax.experimental.pallas import tpu_sc as plsc

N = 10000
E = 320000
IN_CH = 128
HID = 32
HEADS = 8
OUT_CH = 128

BLK = 400
ACC_W = 144


def _lrelu_exp(s):
    return jnp.exp(jnp.maximum(s, 0.2 * s))


def _tc1_body(x_ref, w1_ref, m1a_ref, m1b_ref, h_ref, a_ref, b_ref):
    h = jnp.dot(x_ref[...], w1_ref[...], preferred_element_type=jnp.float32)
    h_ref[...] = h
    a_ref[...] = jnp.dot(h, m1a_ref[...], preferred_element_type=jnp.float32)
    b_ref[...] = jnp.dot(h, m1b_ref[...], preferred_element_type=jnp.float32)


def _tc1(x, W1, M1a, M1b):
    return pl.pallas_call(
        _tc1_body,
        grid=(N // BLK,),
        in_specs=[
            pl.BlockSpec((BLK, IN_CH), lambda i: (i, 0)),
            pl.BlockSpec((IN_CH, HEADS * HID), lambda i: (0, 0)),
            pl.BlockSpec((HEADS * HID, 16), lambda i: (0, 0)),
            pl.BlockSpec((HEADS * HID, 16), lambda i: (0, 0)),
        ],
        out_specs=[
            pl.BlockSpec((BLK, HEADS * HID), lambda i: (i, 0)),
            pl.BlockSpec((BLK, 16), lambda i: (i, 0)),
            pl.BlockSpec((BLK, 16), lambda i: (i, 0)),
        ],
        out_shape=[
            jax.ShapeDtypeStruct((N, HEADS * HID), jnp.float32),
            jax.ShapeDtypeStruct((N, 16), jnp.float32),
            jax.ShapeDtypeStruct((N, 16), jnp.float32),
        ],
    )(x, W1, M1a, M1b)


def _tc2_body(accA_ref, accB_ref, asdA_ref, h1_ref, b1_ref, w2_ref,
              e8_ref, p2a_ref, p2b_ref, h2_ref, a2_ref, b2t_ref):
    asd = asdA_ref[...]
    ws = _lrelu_exp(asd[:, :8] + asd[:, 8:16])
    accA = accA_ref[...]
    accB = accB_ref[...]
    numer = jnp.concatenate([accA[:, :128], accB[:, :128]], axis=1)
    den8 = accA[:, 128:136] + ws
    e8 = e8_ref[...]
    numer = numer + jnp.dot(ws, e8, preferred_element_type=jnp.float32) * h1_ref[...]
    dene = jnp.dot(den8, e8, preferred_element_type=jnp.float32)
    o1 = numer / dene + b1_ref[...]
    g = jnp.where(o1 > 0, o1, jnp.exp(jnp.minimum(o1, 0.0)) - 1.0)
    h2 = jnp.dot(g, w2_ref[...], preferred_element_type=jnp.float32)
    h2_ref[...] = h2
    a2_ref[...] = jnp.dot(h2, p2a_ref[...], preferred_element_type=jnp.float32)
    b2t_ref[...] = jnp.dot(h2, p2b_ref[...], preferred_element_type=jnp.float32)


def _tc2(accA, accB, asdA, h1, b1, W2, E8, P2a, P2b):
    return pl.pallas_call(
        _tc2_body,
        grid=(N // BLK,),
        in_specs=[
            pl.BlockSpec((BLK, ACC_W), lambda i: (i, 0)),
            pl.BlockSpec((BLK, ACC_W), lambda i: (i, 0)),
            pl.BlockSpec((BLK, 16), lambda i: (i, 0)),
            pl.BlockSpec((BLK, HEADS * HID), lambda i: (i, 0)),
            pl.BlockSpec((1, HEADS * HID), lambda i: (0, 0)),
            pl.BlockSpec((HEADS * HID, OUT_CH), lambda i: (0, 0)),
            pl.BlockSpec((HEADS, HEADS * HID), lambda i: (0, 0)),
            pl.BlockSpec((OUT_CH, 16), lambda i: (0, 0)),
            pl.BlockSpec((OUT_CH, 16), lambda i: (0, 0)),
        ],
        out_specs=[
            pl.BlockSpec((BLK, OUT_CH), lambda i: (i, 0)),
            pl.BlockSpec((BLK, 16), lambda i: (i, 0)),
            pl.BlockSpec((BLK, 16), lambda i: (i, 0)),
        ],
        out_shape=[
            jax.ShapeDtypeStruct((N, OUT_CH), jnp.float32),
            jax.ShapeDtypeStruct((N, 16), jnp.float32),
            jax.ShapeDtypeStruct((N, 16), jnp.float32),
        ],
    )(accA, accB, asdA, h1, b1, W2, E8, P2a, P2b)


def _tc3_body(accA_ref, accB_ref, asd2A_ref, asd2B_ref, h2_ref, b2_ref, out_ref):
    s2 = asd2A_ref[...][:, 0:1] + asd2B_ref[...][:, 0:1]
    ws2 = _lrelu_exp(s2)
    accA = accA_ref[...]
    accB = accB_ref[...]
    numer = accA[:, :128] + accB[:, :128] + ws2 * h2_ref[...]
    den = accA[:, 128:129] + accB[:, 128:129] + ws2
    out_ref[...] = numer / den + b2_ref[...]


def _tc3(accA, accB, asd2A, asd2B, h2, b2):
    return pl.pallas_call(
        _tc3_body,
        grid=(N // BLK,),
        in_specs=[
            pl.BlockSpec((BLK, ACC_W), lambda i: (i, 0)),
            pl.BlockSpec((BLK, ACC_W), lambda i: (i, 0)),
            pl.BlockSpec((BLK, 16), lambda i: (i, 0)),
            pl.BlockSpec((BLK, 16), lambda i: (i, 0)),
            pl.BlockSpec((BLK, OUT_CH), lambda i: (i, 0)),
            pl.BlockSpec((1, OUT_CH), lambda i: (0, 0)),
        ],
        out_specs=pl.BlockSpec((BLK, OUT_CH), lambda i: (i, 0)),
        out_shape=jax.ShapeDtypeStruct((N, OUT_CH), jnp.float32),
    )(accA, accB, asd2A, asd2B, h2, b2)


NPAD = 10240
R_T = 640
ACC_R = 648
C_SCAN = 512
NSCAN = E // C_SCAN
B_E = 64


def _dyn_gather16(x, idx):
    return lax.gather(
        x, idx[:, None],
        lax.GatherDimensionNumbers(offset_dims=(), collapsed_slice_dims=(0,),
                                   start_index_map=(0,)),
        (1,), mode=lax.GatherScatterMode.PROMISE_IN_BOUNDS)


def _sc_layer1(h1r, asdA, asdB, src, dst):
    mesh = plsc.VectorSubcoreMesh(core_axis_name="c", subcore_axis_name="s")

    @functools.partial(
        pl.kernel,
        out_type=jax.ShapeDtypeStruct((2, NPAD, ACC_W), jnp.float32),
        mesh=mesh,
        compiler_params=pltpu.CompilerParams(use_tc_tiling_on_sc=False,
                                            needs_layout_passes=False),
        scratch_types=[
            pltpu.VMEM((C_SCAN,), jnp.int32),
            pltpu.VMEM((C_SCAN,), jnp.int32),
            pltpu.VMEM((640,), jnp.int32),
            pltpu.VMEM((640,), jnp.int32),
            pltpu.VMEM((B_E,), jnp.int32),
            pltpu.VMEM((B_E,), jnp.int32),
            pltpu.VMEM((B_E,), jnp.int32),
            pltpu.VMEM((B_E, 16), jnp.float32),
            pltpu.VMEM((B_E, 16), jnp.float32),
            pltpu.VMEM((B_E, 128), jnp.float32),
            pltpu.VMEM((ACC_R, ACC_W), jnp.float32),
            pltpu.SemaphoreType.DMA,
            pltpu.SemaphoreType.DMA,
            pltpu.SemaphoreType.DMA,
        ],
    )
    def k(h1r_hbm, asdA_hbm, asdB_hbm, src_hbm, dst_hbm, out_hbm,
          dbuf, sbufi, clist_s, clist_d, sidx, idx2, idxg, abuf, bbuf, hbuf,
          acc, sem1, sem2, sem3):
        c = lax.axis_index("c")
        s = lax.axis_index("s")
        lo = s * R_T
        hi = lo + R_T
        lane = lax.iota(jnp.int32, 16)
        msk8 = lane < 8
        zv = jnp.zeros((16,), jnp.float32)
        sp_idx = [jnp.zeros((16,), jnp.int32) + (4 * c + m) for m in range(4)]
        zi16 = jnp.zeros((16,), jnp.int32)

        def zero_row(r, carry):
            for j in range(ACC_W // 16):
                acc[r, pl.ds(16 * j, 16)] = zv
            return carry

        lax.fori_loop(0, ACC_R, zero_row, 0)

        def do_batch(boff):
            boff = pl.multiple_of(boff, 16)
            for j in range(B_E // 16):
                sv = clist_s[pl.ds(boff + 16 * j, 16)]
                sidx[pl.ds(16 * j, 16)] = sv
                idx2[pl.ds(16 * j, 16)] = sv * 2 + c
                idxg[pl.ds(16 * j, 16)] = clist_d[pl.ds(boff + 16 * j, 16)] + lo
            cp1 = pltpu.async_copy(asdA_hbm.at[sidx], abuf, sem1)
            cp2 = pltpu.async_copy(asdB_hbm.at[idxg], bbuf, sem2)
            cp3 = pltpu.async_copy(h1r_hbm.at[idx2], hbuf, sem3)
            cp1.wait()
            cp2.wait()
            cp3.wait()

            def edge(kk, _carry):
                sv16 = abuf[kk] + bbuf[kk]
                w = jnp.exp(jnp.maximum(sv16, 0.2 * sv16))
                dv = clist_d[pl.ds(boff + kk, 16)]
                rowv = _dyn_gather16(dv, zi16)
                plsc.addupdate_scatter(
                    acc, [rowv, 128 + lane], jnp.where(msk8, w, 0.0))
                for m in range(4):
                    sp = _dyn_gather16(w, sp_idx[m])
                    plsc.addupdate_scatter(
                        acc, [rowv, 32 * m + lane],
                        hbuf[kk, pl.ds(32 * m, 16)] * sp)
                    plsc.addupdate_scatter(
                        acc, [rowv, 32 * m + 16 + lane],
                        hbuf[kk, pl.ds(32 * m + 16, 16)] * sp)
                return _carry

            lax.fori_loop(0, B_E, edge, 0)

        def chunk_body(i, fill):
            off = i * C_SCAN
            pltpu.sync_copy(dst_hbm.at[pl.ds(off, C_SCAN)], dbuf)
            pltpu.sync_copy(src_hbm.at[pl.ds(off, C_SCAN)], sbufi)

            def scan_v(v, f):
                d = dbuf[pl.ds(v * 16, 16)]
                m = (d >= lo) & (d < hi)
                mi = m.astype(jnp.int32)
                ps = plsc.cumsum(mi)
                pos = ps - mi + f
                plsc.store_scatter(clist_d, [pos], d - lo, mask=m)
                plsc.store_scatter(
                    clist_s, [pos], sbufi[pl.ds(v * 16, 16)], mask=m)
                return f + plsc.all_reduce_population_count(m)[0]

            total = lax.fori_loop(0, C_SCAN // 16, scan_v, fill)
            nb = total // B_E

            def batch_loop(b, carry):
                do_batch(b * B_E)
                return carry

            lax.fori_loop(0, nb, batch_loop, 0)
            left = total - nb * B_E

            mb = pl.multiple_of(nb * B_E, 16)
            for j in range(B_E // 16):
                vd = clist_d[pl.ds(mb + 16 * j, 16)]
                vs = clist_s[pl.ds(mb + 16 * j, 16)]
                clist_d[pl.ds(16 * j, 16)] = vd
                clist_s[pl.ds(16 * j, 16)] = vs
            return left

        fill = lax.fori_loop(0, NSCAN, chunk_body, jnp.int32(0))

        @pl.when(fill > 0)
        def _():
            for j in range(B_E // 16):
                idxl = 16 * j + lane
                m = idxl < fill
                vd = clist_d[pl.ds(16 * j, 16)]
                vs = clist_s[pl.ds(16 * j, 16)]
                clist_d[pl.ds(16 * j, 16)] = jnp.where(m, vd, R_T)
                clist_s[pl.ds(16 * j, 16)] = jnp.where(m, vs, 0)
            do_batch(0)

        pltpu.sync_copy(acc.at[pl.ds(0, R_T)], out_hbm.at[c, pl.ds(lo, R_T)])

    return k(h1r, asdA, asdB, src, dst)


def _sc_layer2(h2, asd2A, asd2B, src, dst):
    mesh = plsc.VectorSubcoreMesh(core_axis_name="c", subcore_axis_name="s")

    @functools.partial(
        pl.kernel,
        out_type=jax.ShapeDtypeStruct((2, NPAD, ACC_W), jnp.float32),
        mesh=mesh,
        compiler_params=pltpu.CompilerParams(use_tc_tiling_on_sc=False,
                                            needs_layout_passes=False),
        scratch_types=[
            pltpu.VMEM((C_SCAN,), jnp.int32),
            pltpu.VMEM((C_SCAN,), jnp.int32),
            pltpu.VMEM((640,), jnp.int32),
            pltpu.VMEM((640,), jnp.int32),
            pltpu.VMEM((B_E,), jnp.int32),
            pltpu.VMEM((B_E,), jnp.int32),
            pltpu.VMEM((B_E, 16), jnp.float32),
            pltpu.VMEM((B_E, 16), jnp.float32),
            pltpu.VMEM((B_E, 128), jnp.float32),
            pltpu.VMEM((ACC_R, ACC_W), jnp.float32),
            pltpu.SemaphoreType.DMA,
            pltpu.SemaphoreType.DMA,
            pltpu.SemaphoreType.DMA,
        ],
    )
    def k(h2_hbm, asdA_hbm, asdB_hbm, src_hbm, dst_hbm, out_hbm,
          dbuf, sbufi, clist_s, clist_d, sidx, idxg, abuf, bbuf, hbuf,
          acc, sem1, sem2, sem3):
        c = lax.axis_index("c")
        s = lax.axis_index("s")
        lo = s * R_T
        hi = lo + R_T
        lane = lax.iota(jnp.int32, 16)
        msk0 = lane < 1
        zv = jnp.zeros((16,), jnp.float32)
        sp0 = jnp.zeros((16,), jnp.int32)

        def zero_row(r, carry):
            for j in range(ACC_W // 16):
                acc[r, pl.ds(16 * j, 16)] = zv
            return carry

        lax.fori_loop(0, ACC_R, zero_row, 0)

        def do_batch(boff):
            boff = pl.multiple_of(boff, 16)
            for j in range(B_E // 16):
                sidx[pl.ds(16 * j, 16)] = clist_s[pl.ds(boff + 16 * j, 16)]
                idxg[pl.ds(16 * j, 16)] = clist_d[pl.ds(boff + 16 * j, 16)] + lo
            cp1 = pltpu.async_copy(asdA_hbm.at[sidx], abuf, sem1)
            cp2 = pltpu.async_copy(asdB_hbm.at[idxg], bbuf, sem2)
            cp3 = pltpu.async_copy(h2_hbm.at[sidx], hbuf, sem3)
            cp1.wait()
            cp2.wait()
            cp3.wait()

            def edge(kk, _carry):
                sv16 = abuf[kk] + bbuf[kk]
                w = jnp.exp(jnp.maximum(sv16, 0.2 * sv16))
                dv = clist_d[pl.ds(boff + kk, 16)]
                rowv = _dyn_gather16(dv, sp0)
                plsc.addupdate_scatter(
                    acc, [rowv, 128 + lane], jnp.where(msk0, w, 0.0))
                sp = _dyn_gather16(w, sp0)
                for j in range(8):
                    plsc.addupdate_scatter(
                        acc, [rowv, 16 * j + lane],
                        hbuf[kk, pl.ds(16 * j, 16)] * sp)
                return _carry

            lax.fori_loop(0, B_E, edge, 0)

        def chunk_body(i, fill):
            cid = i * 2 + c
            off = cid * C_SCAN
            pltpu.sync_copy(dst_hbm.at[pl.ds(off, C_SCAN)], dbuf)
            pltpu.sync_copy(src_hbm.at[pl.ds(off, C_SCAN)], sbufi)

            def scan_v(v, f):
                d = dbuf[pl.ds(v * 16, 16)]
                m = (d >= lo) & (d < hi)
                mi = m.astype(jnp.int32)
                ps = plsc.cumsum(mi)
                pos = ps - mi + f
                plsc.store_scatter(clist_d, [pos], d - lo, mask=m)
                plsc.store_scatter(
                    clist_s, [pos], sbufi[pl.ds(v * 16, 16)], mask=m)
                return f + plsc.all_reduce_population_count(m)[0]

            total = lax.fori_loop(0, C_SCAN // 16, scan_v, fill)
            nb = total // B_E

            def batch_loop(b, carry):
                do_batch(b * B_E)
                return carry

            lax.fori_loop(0, nb, batch_loop, 0)
            left = total - nb * B_E
            mb = pl.multiple_of(nb * B_E, 16)
            for j in range(B_E // 16):
                vd = clist_d[pl.ds(mb + 16 * j, 16)]
                vs = clist_s[pl.ds(mb + 16 * j, 16)]
                clist_d[pl.ds(16 * j, 16)] = vd
                clist_s[pl.ds(16 * j, 16)] = vs
            return left

        nchunks_c = (NSCAN + 1 - c) // 2
        fill = lax.fori_loop(0, nchunks_c, chunk_body, jnp.int32(0))

        @pl.when(fill > 0)
        def _():
            for j in range(B_E // 16):
                idxl = 16 * j + lane
                m = idxl < fill
                vd = clist_d[pl.ds(16 * j, 16)]
                vs = clist_s[pl.ds(16 * j, 16)]
                clist_d[pl.ds(16 * j, 16)] = jnp.where(m, vd, R_T)
                clist_s[pl.ds(16 * j, 16)] = jnp.where(m, vs, 0)
            do_batch(0)

        pltpu.sync_copy(acc.at[pl.ds(0, R_T)], out_hbm.at[c, pl.ds(lo, R_T)])

    return k(h2, asd2A, asd2B, src, dst)


def kernel(x, edge_index, W1, a_src1, a_dst1, b1, W2, a_src2, a_dst2, b2):
    src = edge_index[0]
    dst = edge_index[1]

    head_of = jnp.arange(HEADS * HID) // HID
    oh = (head_of[:, None] == jnp.arange(HEADS)[None, :]).astype(jnp.float32)
    A1s = a_src1.reshape(-1)[:, None] * oh
    A1d = a_dst1.reshape(-1)[:, None] * oh
    z8 = jnp.zeros((HEADS * HID, 8), jnp.float32)
    M1a = jnp.concatenate([A1s, A1d], axis=1)
    M1b = jnp.concatenate([A1d, z8], axis=1)

    E8 = jnp.repeat(jnp.eye(HEADS, dtype=jnp.float32), HID, axis=1)
    P2a = jnp.concatenate([a_src2.reshape(OUT_CH, 1),
                           jnp.zeros((OUT_CH, 15), jnp.float32)], axis=1)
    P2b = jnp.concatenate([a_dst2.reshape(OUT_CH, 1),
                           jnp.zeros((OUT_CH, 15), jnp.float32)], axis=1)

    h1, asdA, asdB = _tc1(x, W1, M1a, M1b)

    h1r = h1.reshape(2 * N, 128)
    acc1 = _sc_layer1(h1r, asdA, asdB, src, dst)

    h2, asd2A, asd2B = _tc2(acc1[0], acc1[1], asdA, h1, b1.reshape(1, -1), W2,
                            E8, P2a, P2b)

    acc2 = _sc_layer2(h2, asd2A, asd2B, src, dst)

    return _tc3(acc2[0], acc2[1], asd2A, asd2B, h2, b2.reshape(1, -1))


# speedup vs baseline: 1.4673x; 1.4673x over previous
import functools

import jax
import jax.numpy as jnp
from jax import lax
from jax.experimental import pallas as pl
from jax.experimental.pallas import tpu as pltpu
from jax.experimental.pallas import tpu_sc as plsc

N = 10000
E = 320000
IN_CH = 128
HID = 32
HEADS = 8
OUT_CH = 128

BLK = 400
ACC_W = 144


def _lrelu_exp(s):
    return jnp.exp(jnp.maximum(s, 0.2 * s))


def _tc1_body(x_ref, w1_ref, m1a_ref, m1b_ref, h_ref, a_ref, b_ref):
    h = jnp.dot(x_ref[...], w1_ref[...], preferred_element_type=jnp.float32)
    h_ref[...] = h
    a_ref[...] = jnp.dot(h, m1a_ref[...], preferred_element_type=jnp.float32)
    b_ref[...] = jnp.dot(h, m1b_ref[...], preferred_element_type=jnp.float32)


def _tc1(x, W1, M1a, M1b):
    return pl.pallas_call(
        _tc1_body,
        grid=(N // BLK,),
        in_specs=[
            pl.BlockSpec((BLK, IN_CH), lambda i: (i, 0)),
            pl.BlockSpec((IN_CH, HEADS * HID), lambda i: (0, 0)),
            pl.BlockSpec((HEADS * HID, 16), lambda i: (0, 0)),
            pl.BlockSpec((HEADS * HID, 16), lambda i: (0, 0)),
        ],
        out_specs=[
            pl.BlockSpec((BLK, HEADS * HID), lambda i: (i, 0)),
            pl.BlockSpec((BLK, 16), lambda i: (i, 0)),
            pl.BlockSpec((BLK, 16), lambda i: (i, 0)),
        ],
        out_shape=[
            jax.ShapeDtypeStruct((N, HEADS * HID), jnp.float32),
            jax.ShapeDtypeStruct((N, 16), jnp.float32),
            jax.ShapeDtypeStruct((N, 16), jnp.float32),
        ],
    )(x, W1, M1a, M1b)


def _tc2_body(accA_ref, accB_ref, asdA_ref, h1_ref, b1_ref, w2_ref,
              e8_ref, p2a_ref, p2b_ref, h2_ref, a2_ref, b2t_ref):
    asd = asdA_ref[...]
    ws = _lrelu_exp(asd[:, :8] + asd[:, 8:16])
    accA = accA_ref[...]
    accB = accB_ref[...]
    numer = jnp.concatenate([accA[:, :128], accB[:, :128]], axis=1)
    den8 = accA[:, 128:136] + ws
    e8 = e8_ref[...]
    numer = numer + jnp.dot(ws, e8, preferred_element_type=jnp.float32) * h1_ref[...]
    dene = jnp.dot(den8, e8, preferred_element_type=jnp.float32)
    o1 = numer / dene + b1_ref[...]
    g = jnp.where(o1 > 0, o1, jnp.exp(jnp.minimum(o1, 0.0)) - 1.0)
    h2 = jnp.dot(g, w2_ref[...], preferred_element_type=jnp.float32)
    h2_ref[...] = h2
    a2_ref[...] = jnp.dot(h2, p2a_ref[...], preferred_element_type=jnp.float32)
    b2t_ref[...] = jnp.dot(h2, p2b_ref[...], preferred_element_type=jnp.float32)


def _tc2(accA, accB, asdA, h1, b1, W2, E8, P2a, P2b):
    return pl.pallas_call(
        _tc2_body,
        grid=(N // BLK,),
        in_specs=[
            pl.BlockSpec((BLK, ACC_W), lambda i: (i, 0)),
            pl.BlockSpec((BLK, ACC_W), lambda i: (i, 0)),
            pl.BlockSpec((BLK, 16), lambda i: (i, 0)),
            pl.BlockSpec((BLK, HEADS * HID), lambda i: (i, 0)),
            pl.BlockSpec((1, HEADS * HID), lambda i: (0, 0)),
            pl.BlockSpec((HEADS * HID, OUT_CH), lambda i: (0, 0)),
            pl.BlockSpec((HEADS, HEADS * HID), lambda i: (0, 0)),
            pl.BlockSpec((OUT_CH, 16), lambda i: (0, 0)),
            pl.BlockSpec((OUT_CH, 16), lambda i: (0, 0)),
        ],
        out_specs=[
            pl.BlockSpec((BLK, OUT_CH), lambda i: (i, 0)),
            pl.BlockSpec((BLK, 16), lambda i: (i, 0)),
            pl.BlockSpec((BLK, 16), lambda i: (i, 0)),
        ],
        out_shape=[
            jax.ShapeDtypeStruct((N, OUT_CH), jnp.float32),
            jax.ShapeDtypeStruct((N, 16), jnp.float32),
            jax.ShapeDtypeStruct((N, 16), jnp.float32),
        ],
    )(accA, accB, asdA, h1, b1, W2, E8, P2a, P2b)


def _tc3_body(accA_ref, accB_ref, asd2A_ref, asd2B_ref, h2_ref, b2_ref, out_ref):
    s2 = asd2A_ref[...][:, 0:1] + asd2B_ref[...][:, 0:1]
    ws2 = _lrelu_exp(s2)
    accA = accA_ref[...]
    accB = accB_ref[...]
    numer = accA[:, :128] + accB[:, :128] + ws2 * h2_ref[...]
    den = accA[:, 128:129] + accB[:, 128:129] + ws2
    out_ref[...] = numer / den + b2_ref[...]


def _tc3(accA, accB, asd2A, asd2B, h2, b2):
    return pl.pallas_call(
        _tc3_body,
        grid=(N // BLK,),
        in_specs=[
            pl.BlockSpec((BLK, ACC_W), lambda i: (i, 0)),
            pl.BlockSpec((BLK, ACC_W), lambda i: (i, 0)),
            pl.BlockSpec((BLK, 16), lambda i: (i, 0)),
            pl.BlockSpec((BLK, 16), lambda i: (i, 0)),
            pl.BlockSpec((BLK, OUT_CH), lambda i: (i, 0)),
            pl.BlockSpec((1, OUT_CH), lambda i: (0, 0)),
        ],
        out_specs=pl.BlockSpec((BLK, OUT_CH), lambda i: (i, 0)),
        out_shape=jax.ShapeDtypeStruct((N, OUT_CH), jnp.float32),
    )(accA, accB, asd2A, asd2B, h2, b2)


NPAD = 10240
R_T = 640
ACC_R = 648
C_SCAN = 512
NSCAN = E // C_SCAN
B_E = 64


def _dyn_gather16(x, idx):
    return lax.gather(
        x, idx[:, None],
        lax.GatherDimensionNumbers(offset_dims=(), collapsed_slice_dims=(0,),
                                   start_index_map=(0,)),
        (1,), mode=lax.GatherScatterMode.PROMISE_IN_BOUNDS)


def _sc_layer1(h1r, asdA, asdB, src, dst):
    mesh = plsc.VectorSubcoreMesh(core_axis_name="c", subcore_axis_name="s")

    @functools.partial(
        pl.kernel,
        out_type=jax.ShapeDtypeStruct((2, NPAD, ACC_W), jnp.float32),
        mesh=mesh,
        compiler_params=pltpu.CompilerParams(use_tc_tiling_on_sc=False,
                                            needs_layout_passes=False),
        scratch_types=[
            pltpu.VMEM((C_SCAN,), jnp.int32),
            pltpu.VMEM((C_SCAN,), jnp.int32),
            pltpu.VMEM((640,), jnp.int32),
            pltpu.VMEM((640,), jnp.int32),
            pltpu.VMEM((B_E,), jnp.int32),
            pltpu.VMEM((B_E,), jnp.int32),
            pltpu.VMEM((B_E,), jnp.int32),
            pltpu.VMEM((B_E, 16), jnp.float32),
            pltpu.VMEM((B_E, 16), jnp.float32),
            pltpu.VMEM((B_E, 128), jnp.float32),
            pltpu.VMEM((ACC_R, ACC_W), jnp.float32),
            pltpu.SemaphoreType.DMA,
            pltpu.SemaphoreType.DMA,
            pltpu.SemaphoreType.DMA,
        ],
    )
    def k(h1r_hbm, asdA_hbm, asdB_hbm, src_hbm, dst_hbm, out_hbm,
          dbuf, sbufi, clist_s, clist_d, sidx, idx2, idxg, abuf, bbuf, hbuf,
          acc, sem1, sem2, sem3):
        c = lax.axis_index("c")
        s = lax.axis_index("s")
        lo = s * R_T
        hi = lo + R_T
        lane = lax.iota(jnp.int32, 16)
        msk8 = lane < 8
        zv = jnp.zeros((16,), jnp.float32)
        sp_idx = [jnp.zeros((16,), jnp.int32) + (4 * c + m) for m in range(4)]
        zi16 = jnp.zeros((16,), jnp.int32)

        def zero_row(r, carry):
            for j in range(ACC_W // 16):
                acc[r, pl.ds(16 * j, 16)] = zv
            return carry

        lax.fori_loop(0, ACC_R, zero_row, 0)

        def do_batch(boff):
            boff = pl.multiple_of(boff, 16)
            for j in range(B_E // 16):
                sv = clist_s[pl.ds(boff + 16 * j, 16)]
                sidx[pl.ds(16 * j, 16)] = sv
                idx2[pl.ds(16 * j, 16)] = sv * 2 + c
                idxg[pl.ds(16 * j, 16)] = clist_d[pl.ds(boff + 16 * j, 16)] + lo
            cp1 = pltpu.async_copy(asdA_hbm.at[sidx], abuf, sem1)
            cp2 = pltpu.async_copy(asdB_hbm.at[idxg], bbuf, sem2)
            cp3 = pltpu.async_copy(h1r_hbm.at[idx2], hbuf, sem3)
            cp1.wait()
            cp2.wait()
            cp3.wait()

            @plsc.parallel_loop(0, B_E, unroll=4)
            def edge(kk):
                sv16 = abuf[kk] + bbuf[kk]
                w = jnp.exp(jnp.maximum(sv16, 0.2 * sv16))
                dv = clist_d[pl.ds(boff + kk, 16)]
                rowv = _dyn_gather16(dv, zi16)
                plsc.addupdate_scatter(
                    acc, [rowv, 128 + lane], jnp.where(msk8, w, 0.0))
                for m in range(4):
                    sp = _dyn_gather16(w, sp_idx[m])
                    plsc.addupdate_scatter(
                        acc, [rowv, 32 * m + lane],
                        hbuf[kk, pl.ds(32 * m, 16)] * sp)
                    plsc.addupdate_scatter(
                        acc, [rowv, 32 * m + 16 + lane],
                        hbuf[kk, pl.ds(32 * m + 16, 16)] * sp)

        def chunk_body(i, fill):
            off = i * C_SCAN
            pltpu.sync_copy(dst_hbm.at[pl.ds(off, C_SCAN)], dbuf)
            pltpu.sync_copy(src_hbm.at[pl.ds(off, C_SCAN)], sbufi)

            def scan_v(v, f):
                d = dbuf[pl.ds(v * 16, 16)]
                m = (d >= lo) & (d < hi)
                mi = m.astype(jnp.int32)
                ps = plsc.cumsum(mi)
                pos = ps - mi + f
                plsc.store_scatter(clist_d, [pos], d - lo, mask=m)
                plsc.store_scatter(
                    clist_s, [pos], sbufi[pl.ds(v * 16, 16)], mask=m)
                return f + plsc.all_reduce_population_count(m)[0]

            total = lax.fori_loop(0, C_SCAN // 16, scan_v, fill)
            nb = total // B_E

            def batch_loop(b, carry):
                do_batch(b * B_E)
                return carry

            lax.fori_loop(0, nb, batch_loop, 0)
            left = total - nb * B_E

            mb = pl.multiple_of(nb * B_E, 16)
            for j in range(B_E // 16):
                vd = clist_d[pl.ds(mb + 16 * j, 16)]
                vs = clist_s[pl.ds(mb + 16 * j, 16)]
                clist_d[pl.ds(16 * j, 16)] = vd
                clist_s[pl.ds(16 * j, 16)] = vs
            return left

        fill = lax.fori_loop(0, NSCAN, chunk_body, jnp.int32(0))

        @pl.when(fill > 0)
        def _():
            for j in range(B_E // 16):
                idxl = 16 * j + lane
                m = idxl < fill
                vd = clist_d[pl.ds(16 * j, 16)]
                vs = clist_s[pl.ds(16 * j, 16)]
                clist_d[pl.ds(16 * j, 16)] = jnp.where(m, vd, R_T)
                clist_s[pl.ds(16 * j, 16)] = jnp.where(m, vs, 0)
            do_batch(0)

        pltpu.sync_copy(acc.at[pl.ds(0, R_T)], out_hbm.at[c, pl.ds(lo, R_T)])

    return k(h1r, asdA, asdB, src, dst)


def _sc_layer2(h2, asd2A, asd2B, src, dst):
    mesh = plsc.VectorSubcoreMesh(core_axis_name="c", subcore_axis_name="s")

    @functools.partial(
        pl.kernel,
        out_type=jax.ShapeDtypeStruct((2, NPAD, ACC_W), jnp.float32),
        mesh=mesh,
        compiler_params=pltpu.CompilerParams(use_tc_tiling_on_sc=False,
                                            needs_layout_passes=False),
        scratch_types=[
            pltpu.VMEM((C_SCAN,), jnp.int32),
            pltpu.VMEM((C_SCAN,), jnp.int32),
            pltpu.VMEM((640,), jnp.int32),
            pltpu.VMEM((640,), jnp.int32),
            pltpu.VMEM((B_E,), jnp.int32),
            pltpu.VMEM((B_E,), jnp.int32),
            pltpu.VMEM((B_E, 16), jnp.float32),
            pltpu.VMEM((B_E, 16), jnp.float32),
            pltpu.VMEM((B_E, 128), jnp.float32),
            pltpu.VMEM((ACC_R, ACC_W), jnp.float32),
            pltpu.SemaphoreType.DMA,
            pltpu.SemaphoreType.DMA,
            pltpu.SemaphoreType.DMA,
        ],
    )
    def k(h2_hbm, asdA_hbm, asdB_hbm, src_hbm, dst_hbm, out_hbm,
          dbuf, sbufi, clist_s, clist_d, sidx, idxg, abuf, bbuf, hbuf,
          acc, sem1, sem2, sem3):
        c = lax.axis_index("c")
        s = lax.axis_index("s")
        lo = s * R_T
        hi = lo + R_T
        lane = lax.iota(jnp.int32, 16)
        msk0 = lane < 1
        zv = jnp.zeros((16,), jnp.float32)
        sp0 = jnp.zeros((16,), jnp.int32)

        def zero_row(r, carry):
            for j in range(ACC_W // 16):
                acc[r, pl.ds(16 * j, 16)] = zv
            return carry

        lax.fori_loop(0, ACC_R, zero_row, 0)

        def do_batch(boff):
            boff = pl.multiple_of(boff, 16)
            for j in range(B_E // 16):
                sidx[pl.ds(16 * j, 16)] = clist_s[pl.ds(boff + 16 * j, 16)]
                idxg[pl.ds(16 * j, 16)] = clist_d[pl.ds(boff + 16 * j, 16)] + lo
            cp1 = pltpu.async_copy(asdA_hbm.at[sidx], abuf, sem1)
            cp2 = pltpu.async_copy(asdB_hbm.at[idxg], bbuf, sem2)
            cp3 = pltpu.async_copy(h2_hbm.at[sidx], hbuf, sem3)
            cp1.wait()
            cp2.wait()
            cp3.wait()

            @plsc.parallel_loop(0, B_E, unroll=4)
            def edge(kk):
                sv16 = abuf[kk] + bbuf[kk]
                w = jnp.exp(jnp.maximum(sv16, 0.2 * sv16))
                dv = clist_d[pl.ds(boff + kk, 16)]
                rowv = _dyn_gather16(dv, sp0)
                plsc.addupdate_scatter(
                    acc, [rowv, 128 + lane], jnp.where(msk0, w, 0.0))
                sp = _dyn_gather16(w, sp0)
                for j in range(8):
                    plsc.addupdate_scatter(
                        acc, [rowv, 16 * j + lane],
                        hbuf[kk, pl.ds(16 * j, 16)] * sp)

        def chunk_body(i, fill):
            cid = i * 2 + c
            off = cid * C_SCAN
            pltpu.sync_copy(dst_hbm.at[pl.ds(off, C_SCAN)], dbuf)
            pltpu.sync_copy(src_hbm.at[pl.ds(off, C_SCAN)], sbufi)

            def scan_v(v, f):
                d = dbuf[pl.ds(v * 16, 16)]
                m = (d >= lo) & (d < hi)
                mi = m.astype(jnp.int32)
                ps = plsc.cumsum(mi)
                pos = ps - mi + f
                plsc.store_scatter(clist_d, [pos], d - lo, mask=m)
                plsc.store_scatter(
                    clist_s, [pos], sbufi[pl.ds(v * 16, 16)], mask=m)
                return f + plsc.all_reduce_population_count(m)[0]

            total = lax.fori_loop(0, C_SCAN // 16, scan_v, fill)
            nb = total // B_E

            def batch_loop(b, carry):
                do_batch(b * B_E)
                return carry

            lax.fori_loop(0, nb, batch_loop, 0)
            left = total - nb * B_E
            mb = pl.multiple_of(nb * B_E, 16)
            for j in range(B_E // 16):
                vd = clist_d[pl.ds(mb + 16 * j, 16)]
                vs = clist_s[pl.ds(mb + 16 * j, 16)]
                clist_d[pl.ds(16 * j, 16)] = vd
                clist_s[pl.ds(16 * j, 16)] = vs
            return left

        nchunks_c = (NSCAN + 1 - c) // 2
        fill = lax.fori_loop(0, nchunks_c, chunk_body, jnp.int32(0))

        @pl.when(fill > 0)
        def _():
            for j in range(B_E // 16):
                idxl = 16 * j + lane
                m = idxl < fill
                vd = clist_d[pl.ds(16 * j, 16)]
                vs = clist_s[pl.ds(16 * j, 16)]
                clist_d[pl.ds(16 * j, 16)] = jnp.where(m, vd, R_T)
                clist_s[pl.ds(16 * j, 16)] = jnp.where(m, vs, 0)
            do_batch(0)

        pltpu.sync_copy(acc.at[pl.ds(0, R_T)], out_hbm.at[c, pl.ds(lo, R_T)])

    return k(h2, asd2A, asd2B, src, dst)


def kernel(x, edge_index, W1, a_src1, a_dst1, b1, W2, a_src2, a_dst2, b2):
    src = edge_index[0]
    dst = edge_index[1]

    head_of = jnp.arange(HEADS * HID) // HID
    oh = (head_of[:, None] == jnp.arange(HEADS)[None, :]).astype(jnp.float32)
    A1s = a_src1.reshape(-1)[:, None] * oh
    A1d = a_dst1.reshape(-1)[:, None] * oh
    z8 = jnp.zeros((HEADS * HID, 8), jnp.float32)
    M1a = jnp.concatenate([A1s, A1d], axis=1)
    M1b = jnp.concatenate([A1d, z8], axis=1)

    E8 = jnp.repeat(jnp.eye(HEADS, dtype=jnp.float32), HID, axis=1)
    P2a = jnp.concatenate([a_src2.reshape(OUT_CH, 1),
                           jnp.zeros((OUT_CH, 15), jnp.float32)], axis=1)
    P2b = jnp.concatenate([a_dst2.reshape(OUT_CH, 1),
                           jnp.zeros((OUT_CH, 15), jnp.float32)], axis=1)

    h1, asdA, asdB = _tc1(x, W1, M1a, M1b)

    h1r = h1.reshape(2 * N, 128)
    acc1 = _sc_layer1(h1r, asdA, asdB, src, dst)

    h2, asd2A, asd2B = _tc2(acc1[0], acc1[1], asdA, h1, b1.reshape(1, -1), W2,
                            E8, P2a, P2b)

    acc2 = _sc_layer2(h2, asd2A, asd2B, src, dst)

    return _tc3(acc2[0], acc2[1], asd2A, asd2B, h2, b2.reshape(1, -1))


# speedup vs baseline: 2.2883x; 1.5595x over previous
import functools

import jax
import jax.numpy as jnp
from jax import lax
from jax.experimental import pallas as pl
from jax.experimental.pallas import tpu as pltpu
from jax.experimental.pallas import tpu_sc as plsc

N = 10000
E = 320000
IN_CH = 128
HID = 32
HEADS = 8
OUT_CH = 128

BLK = 400
ACC_W = 144


def _lrelu_exp(s):
    return jnp.exp(jnp.maximum(s, 0.2 * s))


def _tc1_body(x_ref, w1_ref, m1a_ref, m1b_ref, h_ref, a_ref, b_ref):
    h = jnp.dot(x_ref[...], w1_ref[...], preferred_element_type=jnp.float32)
    h_ref[...] = h
    a_ref[...] = jnp.dot(h, m1a_ref[...], preferred_element_type=jnp.float32)
    b_ref[...] = jnp.dot(h, m1b_ref[...], preferred_element_type=jnp.float32)


def _tc1(x, W1, M1a, M1b):
    return pl.pallas_call(
        _tc1_body,
        grid=(N // BLK,),
        in_specs=[
            pl.BlockSpec((BLK, IN_CH), lambda i: (i, 0)),
            pl.BlockSpec((IN_CH, HEADS * HID), lambda i: (0, 0)),
            pl.BlockSpec((HEADS * HID, 16), lambda i: (0, 0)),
            pl.BlockSpec((HEADS * HID, 16), lambda i: (0, 0)),
        ],
        out_specs=[
            pl.BlockSpec((BLK, HEADS * HID), lambda i: (i, 0)),
            pl.BlockSpec((BLK, 16), lambda i: (i, 0)),
            pl.BlockSpec((BLK, 16), lambda i: (i, 0)),
        ],
        out_shape=[
            jax.ShapeDtypeStruct((N, HEADS * HID), jnp.float32),
            jax.ShapeDtypeStruct((N, 16), jnp.float32),
            jax.ShapeDtypeStruct((N, 16), jnp.float32),
        ],
    )(x, W1, M1a, M1b)


def _tc2_body(accA_ref, accB_ref, asdA_ref, h1_ref, b1_ref, w2_ref,
              e8_ref, p2a_ref, p2b_ref, h2_ref, a2_ref, b2t_ref):
    asd = asdA_ref[...]
    ws = _lrelu_exp(asd[:, :8] + asd[:, 8:16])
    accA = accA_ref[...]
    accB = accB_ref[...]
    numer = jnp.concatenate([accA[:, :128], accB[:, :128]], axis=1)
    den8 = accA[:, 128:136] + ws
    e8 = e8_ref[...]
    numer = numer + jnp.dot(ws, e8, preferred_element_type=jnp.float32) * h1_ref[...]
    dene = jnp.dot(den8, e8, preferred_element_type=jnp.float32)
    o1 = numer / dene + b1_ref[...]
    g = jnp.where(o1 > 0, o1, jnp.exp(jnp.minimum(o1, 0.0)) - 1.0)
    h2 = jnp.dot(g, w2_ref[...], preferred_element_type=jnp.float32)
    h2_ref[...] = h2
    a2_ref[...] = jnp.dot(h2, p2a_ref[...], preferred_element_type=jnp.float32)
    b2t_ref[...] = jnp.dot(h2, p2b_ref[...], preferred_element_type=jnp.float32)


def _tc2(accA, accB, asdA, h1, b1, W2, E8, P2a, P2b):
    return pl.pallas_call(
        _tc2_body,
        grid=(N // BLK,),
        in_specs=[
            pl.BlockSpec((BLK, ACC_W), lambda i: (i, 0)),
            pl.BlockSpec((BLK, ACC_W), lambda i: (i, 0)),
            pl.BlockSpec((BLK, 16), lambda i: (i, 0)),
            pl.BlockSpec((BLK, HEADS * HID), lambda i: (i, 0)),
            pl.BlockSpec((1, HEADS * HID), lambda i: (0, 0)),
            pl.BlockSpec((HEADS * HID, OUT_CH), lambda i: (0, 0)),
            pl.BlockSpec((HEADS, HEADS * HID), lambda i: (0, 0)),
            pl.BlockSpec((OUT_CH, 16), lambda i: (0, 0)),
            pl.BlockSpec((OUT_CH, 16), lambda i: (0, 0)),
        ],
        out_specs=[
            pl.BlockSpec((BLK, OUT_CH), lambda i: (i, 0)),
            pl.BlockSpec((BLK, 16), lambda i: (i, 0)),
            pl.BlockSpec((BLK, 16), lambda i: (i, 0)),
        ],
        out_shape=[
            jax.ShapeDtypeStruct((N, OUT_CH), jnp.float32),
            jax.ShapeDtypeStruct((N, 16), jnp.float32),
            jax.ShapeDtypeStruct((N, 16), jnp.float32),
        ],
    )(accA, accB, asdA, h1, b1, W2, E8, P2a, P2b)


def _tc3_body(accA_ref, accB_ref, asd2A_ref, asd2B_ref, h2_ref, b2_ref, out_ref):
    s2 = asd2A_ref[...][:, 0:1] + asd2B_ref[...][:, 0:1]
    ws2 = _lrelu_exp(s2)
    accA = accA_ref[...]
    accB = accB_ref[...]
    numer = accA[:, :128] + accB[:, :128] + ws2 * h2_ref[...]
    den = accA[:, 128:129] + accB[:, 128:129] + ws2
    out_ref[...] = numer / den + b2_ref[...]


def _tc3(accA, accB, asd2A, asd2B, h2, b2):
    return pl.pallas_call(
        _tc3_body,
        grid=(N // BLK,),
        in_specs=[
            pl.BlockSpec((BLK, ACC_W), lambda i: (i, 0)),
            pl.BlockSpec((BLK, ACC_W), lambda i: (i, 0)),
            pl.BlockSpec((BLK, 16), lambda i: (i, 0)),
            pl.BlockSpec((BLK, 16), lambda i: (i, 0)),
            pl.BlockSpec((BLK, OUT_CH), lambda i: (i, 0)),
            pl.BlockSpec((1, OUT_CH), lambda i: (0, 0)),
        ],
        out_specs=pl.BlockSpec((BLK, OUT_CH), lambda i: (i, 0)),
        out_shape=jax.ShapeDtypeStruct((N, OUT_CH), jnp.float32),
    )(accA, accB, asd2A, asd2B, h2, b2)


NPAD = 10240
R_T = 640
ACC_R = 641
C_SCAN = 512
NSCAN = E // C_SCAN
B_E = 64
CLIST = 576

_SC_PARAMS = pltpu.CompilerParams(use_tc_tiling_on_sc=False,
                                  needs_layout_passes=False)


def _dyn_gather16(x, idx):
    return lax.gather(
        x, idx[:, None],
        lax.GatherDimensionNumbers(offset_dims=(), collapsed_slice_dims=(0,),
                                   start_index_map=(0,)),
        (1,), mode=lax.GatherScatterMode.PROMISE_IN_BOUNDS)


def _sc_layer1(h1r, asdA, asdB, eb):
    mesh = plsc.VectorSubcoreMesh(core_axis_name="c", subcore_axis_name="s")

    @functools.partial(
        pl.kernel,
        out_type=jax.ShapeDtypeStruct((2, NPAD, ACC_W), jnp.float32),
        mesh=mesh,
        compiler_params=_SC_PARAMS,
        scratch_types=[
            [pltpu.VMEM((2, C_SCAN), jnp.int32)] * 2,
            pltpu.VMEM((CLIST,), jnp.int32),
            pltpu.VMEM((CLIST,), jnp.int32),
            [pltpu.VMEM((B_E,), jnp.int32)] * 2,
            [pltpu.VMEM((B_E,), jnp.int32)] * 2,
            [pltpu.VMEM((B_E,), jnp.int32)] * 2,
            [pltpu.VMEM((B_E, 16), jnp.float32)] * 2,
            [pltpu.VMEM((B_E, 16), jnp.float32)] * 2,
            [pltpu.VMEM((B_E, 128), jnp.float32)] * 2,
            pltpu.VMEM((ACC_R, ACC_W), jnp.float32),
            [pltpu.SemaphoreType.DMA] * 2,
            [pltpu.SemaphoreType.DMA] * 2,
            [pltpu.SemaphoreType.DMA] * 2,
            [pltpu.SemaphoreType.DMA] * 2,
        ],
    )
    def k(h1r_hbm, asdA_hbm, asdB_hbm, eb_hbm, out_hbm,
          ebuf, clist_s, clist_d, sidx, idx2, idxg, abuf, bbuf, hbuf,
          acc, semE, semA, semB, semH):
        c = lax.axis_index("c")
        s = lax.axis_index("s")
        lo = s * R_T
        hi = lo + R_T
        lane = lax.iota(jnp.int32, 16)
        msk8 = lane < 8
        zv = jnp.zeros((16,), jnp.float32)
        zi16 = jnp.zeros((16,), jnp.int32)
        sp_idx = [zi16 + (4 * c + m) for m in range(4)]

        def zero_row(r, carry):
            for j in range(ACC_W // 16):
                acc[r, pl.ds(16 * j, 16)] = zv
            return carry

        lax.fori_loop(0, ACC_R, zero_row, 0)

        def issue_batch(b, p):
            boff = pl.multiple_of(b * B_E, 16)
            for j in range(B_E // 16):
                sv = clist_s[pl.ds(boff + 16 * j, 16)]
                sidx[p][pl.ds(16 * j, 16)] = sv
                idx2[p][pl.ds(16 * j, 16)] = sv * 2 + c
                idxg[p][pl.ds(16 * j, 16)] = (
                    clist_d[pl.ds(boff + 16 * j, 16)] + lo)
            pltpu.async_copy(asdA_hbm.at[sidx[p]], abuf[p], semA[p])
            pltpu.async_copy(asdB_hbm.at[idxg[p]], bbuf[p], semB[p])
            pltpu.async_copy(h1r_hbm.at[idx2[p]], hbuf[p], semH[p])

        def consume_batch(b, p):
            boff = pl.multiple_of(b * B_E, 16)
            pltpu.make_async_copy(asdA_hbm.at[sidx[p]], abuf[p], semA[p]).wait()
            pltpu.make_async_copy(asdB_hbm.at[idxg[p]], bbuf[p], semB[p]).wait()
            pltpu.make_async_copy(h1r_hbm.at[idx2[p]], hbuf[p], semH[p]).wait()

            @plsc.parallel_loop(0, B_E, unroll=4)
            def edge(kk):
                sv16 = abuf[p][kk] + bbuf[p][kk]
                w = jnp.exp(jnp.maximum(sv16, 0.2 * sv16))
                dv = clist_d[pl.ds(boff + kk, 16)]
                rowv = _dyn_gather16(dv, zi16)
                plsc.addupdate_scatter(
                    acc, [rowv, 128 + lane], jnp.where(msk8, w, 0.0))
                for m in range(4):
                    sp = _dyn_gather16(w, sp_idx[m])
                    plsc.addupdate_scatter(
                        acc, [rowv, 32 * m + lane],
                        hbuf[p][kk, pl.ds(32 * m, 16)] * sp)
                    plsc.addupdate_scatter(
                        acc, [rowv, 32 * m + 16 + lane],
                        hbuf[p][kk, pl.ds(32 * m + 16, 16)] * sp)

        def issue_scan(i, p):
            pltpu.async_copy(eb_hbm.at[i], ebuf[p], semE[p])

        def process_chunk(i, p, fill):
            pltpu.make_async_copy(eb_hbm.at[i], ebuf[p], semE[p]).wait()

            def scan_v(v, f):
                d = ebuf[p][1, pl.ds(v * 16, 16)]
                m = (d >= lo) & (d < hi)
                mi = m.astype(jnp.int32)
                ps = plsc.cumsum(mi)
                pos = ps - mi + f
                plsc.store_scatter(clist_d, [pos], d - lo, mask=m)
                plsc.store_scatter(
                    clist_s, [pos], ebuf[p][0, pl.ds(v * 16, 16)], mask=m)
                return f + plsc.all_reduce_population_count(m)[0]

            total = lax.fori_loop(0, C_SCAN // 16, scan_v, fill)
            nb = total // B_E

            @pl.when(nb > 0)
            def _():
                issue_batch(0, 0)

            def bpair(q, carry):
                for pb in range(2):
                    b = 2 * q + pb

                    @pl.when(b + 1 < nb)
                    def _():
                        issue_batch(b + 1, 1 - pb)

                    @pl.when(b < nb)
                    def _():
                        consume_batch(b, pb)
                return carry

            lax.fori_loop(0, (nb + 1) // 2, bpair, 0)
            left = total - nb * B_E

            mb = pl.multiple_of(nb * B_E, 16)
            for j in range(B_E // 16):
                vd = clist_d[pl.ds(mb + 16 * j, 16)]
                vs = clist_s[pl.ds(mb + 16 * j, 16)]
                clist_d[pl.ds(16 * j, 16)] = vd
                clist_s[pl.ds(16 * j, 16)] = vs
            return left

        issue_scan(0, 0)

        def pair(q, fill):
            for pb in range(2):
                i = 2 * q + pb

                @pl.when(i + 1 < NSCAN)
                def _():
                    issue_scan(i + 1, 1 - pb)

                fill = lax.cond(
                    i < NSCAN,
                    lambda f: process_chunk(i, pb, f),
                    lambda f: f,
                    fill)
            return fill

        fill = lax.fori_loop(0, (NSCAN + 1) // 2, pair, jnp.int32(0))

        @pl.when(fill > 0)
        def _():
            for j in range(B_E // 16):
                idxl = 16 * j + lane
                m = idxl < fill
                vd = clist_d[pl.ds(16 * j, 16)]
                vs = clist_s[pl.ds(16 * j, 16)]
                clist_d[pl.ds(16 * j, 16)] = jnp.where(m, vd, R_T)
                clist_s[pl.ds(16 * j, 16)] = jnp.where(m, vs, 0)
            issue_batch(0, 0)
            consume_batch(0, 0)

        pltpu.sync_copy(acc.at[pl.ds(0, R_T)], out_hbm.at[c, pl.ds(lo, R_T)])

    return k(h1r, asdA, asdB, eb)


def _sc_layer2(h2, asd2A, asd2B, eb):
    mesh = plsc.VectorSubcoreMesh(core_axis_name="c", subcore_axis_name="s")

    @functools.partial(
        pl.kernel,
        out_type=jax.ShapeDtypeStruct((2, NPAD, ACC_W), jnp.float32),
        mesh=mesh,
        compiler_params=_SC_PARAMS,
        scratch_types=[
            [pltpu.VMEM((2, C_SCAN), jnp.int32)] * 2,
            pltpu.VMEM((CLIST,), jnp.int32),
            pltpu.VMEM((CLIST,), jnp.int32),
            [pltpu.VMEM((B_E,), jnp.int32)] * 2,
            [pltpu.VMEM((B_E,), jnp.int32)] * 2,
            [pltpu.VMEM((B_E, 16), jnp.float32)] * 2,
            [pltpu.VMEM((B_E, 16), jnp.float32)] * 2,
            [pltpu.VMEM((B_E, 128), jnp.float32)] * 2,
            pltpu.VMEM((ACC_R, ACC_W), jnp.float32),
            [pltpu.SemaphoreType.DMA] * 2,
            [pltpu.SemaphoreType.DMA] * 2,
            [pltpu.SemaphoreType.DMA] * 2,
            [pltpu.SemaphoreType.DMA] * 2,
        ],
    )
    def k(h2_hbm, asdA_hbm, asdB_hbm, eb_hbm, out_hbm,
          ebuf, clist_s, clist_d, sidx, idxg, abuf, bbuf, hbuf,
          acc, semE, semA, semB, semH):
        c = lax.axis_index("c")
        s = lax.axis_index("s")
        lo = s * R_T
        hi = lo + R_T
        lane = lax.iota(jnp.int32, 16)
        msk0 = lane < 1
        zv = jnp.zeros((16,), jnp.float32)
        sp0 = jnp.zeros((16,), jnp.int32)

        def zero_row(r, carry):
            for j in range(ACC_W // 16):
                acc[r, pl.ds(16 * j, 16)] = zv
            return carry

        lax.fori_loop(0, ACC_R, zero_row, 0)

        def issue_batch(b, p):
            boff = pl.multiple_of(b * B_E, 16)
            for j in range(B_E // 16):
                sidx[p][pl.ds(16 * j, 16)] = clist_s[pl.ds(boff + 16 * j, 16)]
                idxg[p][pl.ds(16 * j, 16)] = (
                    clist_d[pl.ds(boff + 16 * j, 16)] + lo)
            pltpu.async_copy(asdA_hbm.at[sidx[p]], abuf[p], semA[p])
            pltpu.async_copy(asdB_hbm.at[idxg[p]], bbuf[p], semB[p])
            pltpu.async_copy(h2_hbm.at[sidx[p]], hbuf[p], semH[p])

        def consume_batch(b, p):
            boff = pl.multiple_of(b * B_E, 16)
            pltpu.make_async_copy(asdA_hbm.at[sidx[p]], abuf[p], semA[p]).wait()
            pltpu.make_async_copy(asdB_hbm.at[idxg[p]], bbuf[p], semB[p]).wait()
            pltpu.make_async_copy(h2_hbm.at[sidx[p]], hbuf[p], semH[p]).wait()

            @plsc.parallel_loop(0, B_E, unroll=4)
            def edge(kk):
                sv16 = abuf[p][kk] + bbuf[p][kk]
                w = jnp.exp(jnp.maximum(sv16, 0.2 * sv16))
                dv = clist_d[pl.ds(boff + kk, 16)]
                rowv = _dyn_gather16(dv, sp0)
                plsc.addupdate_scatter(
                    acc, [rowv, 128 + lane], jnp.where(msk0, w, 0.0))
                sp = _dyn_gather16(w, sp0)
                for j in range(8):
                    plsc.addupdate_scatter(
                        acc, [rowv, 16 * j + lane],
                        hbuf[p][kk, pl.ds(16 * j, 16)] * sp)

        def issue_scan(cid, p):
            pltpu.async_copy(eb_hbm.at[cid], ebuf[p], semE[p])

        def process_chunk(cid, p, fill):
            pltpu.make_async_copy(eb_hbm.at[cid], ebuf[p], semE[p]).wait()

            def scan_v(v, f):
                d = ebuf[p][1, pl.ds(v * 16, 16)]
                m = (d >= lo) & (d < hi)
                mi = m.astype(jnp.int32)
                ps = plsc.cumsum(mi)
                pos = ps - mi + f
                plsc.store_scatter(clist_d, [pos], d - lo, mask=m)
                plsc.store_scatter(
                    clist_s, [pos], ebuf[p][0, pl.ds(v * 16, 16)], mask=m)
                return f + plsc.all_reduce_population_count(m)[0]

            total = lax.fori_loop(0, C_SCAN // 16, scan_v, fill)
            nb = total // B_E

            @pl.when(nb > 0)
            def _():
                issue_batch(0, 0)

            def bpair(q, carry):
                for pb in range(2):
                    b = 2 * q + pb

                    @pl.when(b + 1 < nb)
                    def _():
                        issue_batch(b + 1, 1 - pb)

                    @pl.when(b < nb)
                    def _():
                        consume_batch(b, pb)
                return carry

            lax.fori_loop(0, (nb + 1) // 2, bpair, 0)
            left = total - nb * B_E
            mb = pl.multiple_of(nb * B_E, 16)
            for j in range(B_E // 16):
                vd = clist_d[pl.ds(mb + 16 * j, 16)]
                vs = clist_s[pl.ds(mb + 16 * j, 16)]
                clist_d[pl.ds(16 * j, 16)] = vd
                clist_s[pl.ds(16 * j, 16)] = vs
            return left

        nchunks_c = (NSCAN + 1 - c) // 2
        issue_scan(c, 0)

        def pair(q, fill):
            for pb in range(2):
                i = 2 * q + pb

                @pl.when(i + 1 < nchunks_c)
                def _():
                    issue_scan(2 * (i + 1) + c, 1 - pb)

                fill = lax.cond(
                    i < nchunks_c,
                    lambda f: process_chunk(2 * i + c, pb, f),
                    lambda f: f,
                    fill)
            return fill

        fill = lax.fori_loop(0, (NSCAN // 2 + 2) // 2, pair, jnp.int32(0))

        @pl.when(fill > 0)
        def _():
            for j in range(B_E // 16):
                idxl = 16 * j + lane
                m = idxl < fill
                vd = clist_d[pl.ds(16 * j, 16)]
                vs = clist_s[pl.ds(16 * j, 16)]
                clist_d[pl.ds(16 * j, 16)] = jnp.where(m, vd, R_T)
                clist_s[pl.ds(16 * j, 16)] = jnp.where(m, vs, 0)
            issue_batch(0, 0)
            consume_batch(0, 0)

        pltpu.sync_copy(acc.at[pl.ds(0, R_T)], out_hbm.at[c, pl.ds(lo, R_T)])

    return k(h2, asd2A, asd2B, eb)


def kernel(x, edge_index, W1, a_src1, a_dst1, b1, W2, a_src2, a_dst2, b2):
    eb = edge_index.reshape(2, NSCAN, C_SCAN).transpose(1, 0, 2)

    head_of = jnp.arange(HEADS * HID) // HID
    oh = (head_of[:, None] == jnp.arange(HEADS)[None, :]).astype(jnp.float32)
    A1s = a_src1.reshape(-1)[:, None] * oh
    A1d = a_dst1.reshape(-1)[:, None] * oh
    z8 = jnp.zeros((HEADS * HID, 8), jnp.float32)
    M1a = jnp.concatenate([A1s, A1d], axis=1)
    M1b = jnp.concatenate([A1d, z8], axis=1)

    E8 = jnp.repeat(jnp.eye(HEADS, dtype=jnp.float32), HID, axis=1)
    P2a = jnp.concatenate([a_src2.reshape(OUT_CH, 1),
                           jnp.zeros((OUT_CH, 15), jnp.float32)], axis=1)
    P2b = jnp.concatenate([a_dst2.reshape(OUT_CH, 1),
                           jnp.zeros((OUT_CH, 15), jnp.float32)], axis=1)

    h1, asdA, asdB = _tc1(x, W1, M1a, M1b)

    h1r = h1.reshape(2 * N, 128)
    acc1 = _sc_layer1(h1r, asdA, asdB, eb)

    h2, asd2A, asd2B = _tc2(acc1[0], acc1[1], asdA, h1, b1.reshape(1, -1), W2,
                            E8, P2a, P2b)

    acc2 = _sc_layer2(h2, asd2A, asd2B, eb)

    return _tc3(acc2[0], acc2[1], asd2A, asd2B, h2, b2.reshape(1, -1))
